# baseline (speedup 1.0000x reference)
import functools

import jax
import jax.numpy as jnp
from jax import lax
from jax.experimental import pallas as pl
from jax.experimental.pallas import tpu as pltpu
from jax.experimental.pallas import tpu_sc as plsc

_LN_EPS = 1e-5
_CH = 128
_NC = 2
_NS = 16
_KB = 32
_NBUF = 8
_G = 6


def _dense_body(xp_ref, xn_ref, g_ref, bt_ref, wr_ref, wn_ref, b_ref,
                root_ref, y_ref):
    g = g_ref[...]
    bt = bt_ref[...]

    def ln(v):
        mu = jnp.mean(v, axis=-1, keepdims=True)
        var = jnp.mean((v - mu) * (v - mu), axis=-1, keepdims=True)
        return (v - mu) * lax.rsqrt(var + _LN_EPS) * g + bt

    x = jnp.concatenate([ln(xp_ref[...]), ln(xn_ref[...])], axis=1)
    root_ref[...] = (
        jnp.dot(x, wr_ref[...], preferred_element_type=jnp.float32) + b_ref[...]
    )
    y = jnp.dot(x, wn_ref[...], preferred_element_type=jnp.float32)
    d_half = y.shape[1] // 2
    y_ref[0] = y[:, :d_half]
    y_ref[1] = y[:, d_half:]


def _combine_body(root_ref, p0_ref, p1_ref, out_ref):
    agg = jnp.concatenate([p0_ref[...], p1_ref[...]], axis=1)
    out_ref[...] = root_ref[...] + agg


def _make_sc_kernel(n_pad, k, dh):
    rows_per_sub = n_pad // _NS
    nblk = k // _KB
    mesh = plsc.VectorSubcoreMesh(core_axis_name="c", subcore_axis_name="s")

    @functools.partial(
        pl.kernel,
        out_type=jax.ShapeDtypeStruct((_NC, n_pad, dh), jnp.float32),
        mesh=mesh,
        scratch_types=[
            pltpu.VMEM((2, _KB, _CH), jnp.int32),
            pltpu.VMEM((2, _KB, _CH), jnp.int32),
            pltpu.VMEM((_NBUF, _CH, dh), jnp.float32),
            pltpu.VMEM_SHARED((n_pad, dh), jnp.float32),
            pltpu.SemaphoreType.DMA,
            pltpu.SemaphoreType.DMA,
            pltpu.SemaphoreType.DMA,
        ],
        compiler_params=pltpu.CompilerParams(use_tc_tiling_on_sc=False),
    )
    def sc_kernel(y_hbm, ei_hbm, zeros_hbm, out_hbm,
                  src_i, dst_i, rows_v, acc, gsem, ssem, isem):
        c = lax.axis_index("c")
        s = lax.axis_index("s")
        pltpu.sync_copy(ei_hbm.at[0, s, pl.ds(0, _KB)], src_i.at[0])
        pltpu.sync_copy(ei_hbm.at[1, s, pl.ds(0, _KB)], dst_i.at[0])
        row0 = s * rows_per_sub
        pltpu.sync_copy(zeros_hbm.at[pl.ds(row0, rows_per_sub)],
                        acc.at[pl.ds(row0, rows_per_sub)])
        plsc.subcore_barrier()

        def gather(pbuf, row, slot):
            pltpu.async_copy(y_hbm.at[c].at[src_i.at[pbuf, row]],
                             rows_v.at[slot], gsem)

        def scatter(pbuf, row, slot):
            pltpu.async_copy(rows_v.at[slot], acc.at[dst_i.at[pbuf, row]],
                             ssem, add=True)

        def wait_rows(sem):
            pltpu.make_async_copy(zeros_hbm.at[pl.ds(0, _CH)],
                                  rows_v.at[0], sem).wait()

        def wait_idx():
            pltpu.make_async_copy(ei_hbm.at[0, 0, pl.ds(0, _KB)],
                                  src_i.at[0], isem).wait()

        for g in range(_G):
            gather(0, g, g % _NBUF)

        for b in range(nblk):
            pb = b % 2
            if b + 1 < nblk:
                pltpu.async_copy(ei_hbm.at[0, s, pl.ds((b + 1) * _KB, _KB)],
                                 src_i.at[1 - pb], isem)
                pltpu.async_copy(ei_hbm.at[1, s, pl.ds((b + 1) * _KB, _KB)],
                                 dst_i.at[1 - pb], isem)

            def body(jj, carry, b=b, pb=pb):
                j = b * _KB + jj
                @pl.when(j >= 2)
                def _():
                    wait_rows(ssem)
                g_loc = jj + _G
                slot = lax.rem(g_loc, _NBUF)
                if b + 1 < nblk:
                    @pl.when(jj == _KB - _G - 1)
                    def _():
                        wait_idx()
                        wait_idx()
                    cross = g_loc >= _KB
                    gp = jnp.where(cross, 1 - pb, pb)
                    grow = jnp.where(cross, g_loc - _KB, g_loc)
                    gather(gp, grow, slot)
                else:
                    @pl.when(g_loc < _KB)
                    def _():
                        gather(pb, g_loc, slot)
                wait_rows(gsem)
                scatter(pb, jj, lax.rem(jj, _NBUF))
                return carry

            lax.fori_loop(0, _KB, body, 0)

        wait_rows(ssem)
        wait_rows(ssem)
        plsc.subcore_barrier()
        pltpu.sync_copy(acc.at[pl.ds(row0, rows_per_sub)],
                        out_hbm.at[c, pl.ds(row0, rows_per_sub)])

    return sc_kernel


def kernel(x_prev, x_same, x_next, edge_index, ln_gamma, ln_beta,
           W_root, W_neigh, b):
    n, d_prev = x_prev.shape
    d_out = W_root.shape[1]
    dh = d_out // 2
    e = edge_index.shape[1]

    k = -(-e // (_NS * _CH * _KB)) * _KB
    e_pad = _NS * _CH * k
    ny = n + 16
    n_pad = -(-(n + 1) // (_NS * 8)) * (_NS * 8)

    bn = 2000
    grid = (n // bn,)
    root, y = pl.pallas_call(
        _dense_body,
        grid=grid,
        in_specs=[
            pl.BlockSpec((bn, d_prev), lambda i: (i, 0)),
            pl.BlockSpec((bn, d_prev), lambda i: (i, 0)),
            pl.BlockSpec((1, d_prev), lambda i: (0, 0)),
            pl.BlockSpec((1, d_prev), lambda i: (0, 0)),
            pl.BlockSpec(W_root.shape, lambda i: (0, 0)),
            pl.BlockSpec(W_neigh.shape, lambda i: (0, 0)),
            pl.BlockSpec((1, d_out), lambda i: (0, 0)),
        ],
        out_specs=[
            pl.BlockSpec((bn, d_out), lambda i: (i, 0)),
            pl.BlockSpec((2, bn, dh), lambda i: (0, i, 0)),
        ],
        out_shape=[
            jax.ShapeDtypeStruct((n, d_out), jnp.float32),
            jax.ShapeDtypeStruct((2, ny, dh), jnp.float32),
        ],
    )(x_prev, x_next, ln_gamma.reshape(1, -1), ln_beta.reshape(1, -1),
      W_root, W_neigh, b.reshape(1, -1))

    npad_e = e_pad - e
    pad_src = jnp.full((1, npad_e), n, jnp.int32)
    pad_dst = (n + jnp.arange(npad_e, dtype=jnp.int32) % (n_pad - n))[None]
    ei = jnp.concatenate([edge_index, jnp.concatenate([pad_src, pad_dst], 0)],
                         axis=1)
    ei = ei.reshape(2, _NS, k, _CH)
    zeros = jnp.zeros((n_pad, dh), jnp.float32)

    partials = _make_sc_kernel(n_pad, k, dh)(y, ei, zeros)

    p0 = partials[0, :n]
    p1 = partials[1, :n]
    out = pl.pallas_call(
        _combine_body,
        grid=grid,
        in_specs=[
            pl.BlockSpec((bn, d_out), lambda i: (i, 0)),
            pl.BlockSpec((bn, dh), lambda i: (i, 0)),
            pl.BlockSpec((bn, dh), lambda i: (i, 0)),
        ],
        out_specs=pl.BlockSpec((bn, d_out), lambda i: (i, 0)),
        out_shape=jax.ShapeDtypeStruct((n, d_out), jnp.float32),
    )(root, p0, p1)
    return out


# speedup vs baseline: 1.7847x; 1.7847x over previous
import functools

import jax
import jax.numpy as jnp
from jax import lax
from jax.experimental import pallas as pl
from jax.experimental.pallas import tpu as pltpu
from jax.experimental.pallas import tpu_sc as plsc

_LN_EPS = 1e-5
_CH = 128
_NC = 2
_NS = 16


def _dense_body(xp_ref, xn_ref, g_ref, bt_ref, wr_ref, wn_ref, b_ref,
                root_ref, y_ref):
    g = g_ref[...]
    bt = bt_ref[...]

    def ln(v):
        mu = jnp.mean(v, axis=-1, keepdims=True)
        var = jnp.mean((v - mu) * (v - mu), axis=-1, keepdims=True)
        return (v - mu) * lax.rsqrt(var + _LN_EPS) * g + bt

    x = jnp.concatenate([ln(xp_ref[...]), ln(xn_ref[...])], axis=1)
    root_ref[...] = (
        jnp.dot(x, wr_ref[...], preferred_element_type=jnp.float32) + b_ref[...]
    )
    y = jnp.dot(x, wn_ref[...], preferred_element_type=jnp.float32)
    d_half = y.shape[1] // 2
    y_ref[0] = y[:, :d_half]
    y_ref[1] = y[:, d_half:]


def _combine_body(root_ref, p0_ref, p1_ref, out_ref):
    agg = jnp.concatenate([p0_ref[...], p1_ref[...]], axis=1)
    out_ref[...] = root_ref[...] + agg


def _make_sc_kernel(n_pad, k, dh):
    rows_per_sub = n_pad // _NS
    mesh = plsc.VectorSubcoreMesh(core_axis_name="c", subcore_axis_name="s")
    nbuf = 6

    @functools.partial(
        pl.kernel,
        out_type=jax.ShapeDtypeStruct((_NC, n_pad, dh), jnp.float32),
        mesh=mesh,
        scratch_types=[
            pltpu.VMEM((k, _CH), jnp.int32),
            pltpu.VMEM((k, _CH), jnp.int32),
            pltpu.VMEM((nbuf, _CH, dh), jnp.float32),
            pltpu.VMEM_SHARED((n_pad, dh), jnp.float32),
            pltpu.SemaphoreType.DMA,
            pltpu.SemaphoreType.DMA,
        ],
        compiler_params=pltpu.CompilerParams(use_tc_tiling_on_sc=False),
    )
    def sc_kernel(y_hbm, ei_hbm, zeros_hbm, out_hbm,
                  src_v, dst_v, rows_v, acc, gsem, ssem):
        c = lax.axis_index("c")
        s = lax.axis_index("s")
        pltpu.sync_copy(ei_hbm.at[0, s], src_v)
        pltpu.sync_copy(ei_hbm.at[1, s], dst_v)
        row0 = s * rows_per_sub
        pltpu.sync_copy(zeros_hbm.at[pl.ds(row0, rows_per_sub)],
                        acc.at[pl.ds(row0, rows_per_sub)])
        plsc.subcore_barrier()

        def gather(j, slot):
            pltpu.async_copy(y_hbm.at[c].at[src_v.at[j]], rows_v.at[slot],
                             gsem)

        def scatter(j, slot):
            pltpu.async_copy(rows_v.at[slot], acc.at[dst_v.at[j]], ssem,
                             add=True)

        def wait(sem):
            pltpu.make_async_copy(zeros_hbm.at[pl.ds(0, _CH)],
                                  rows_v.at[0], sem).wait()

        for j in range(min(4, k)):
            gather(j, j % nbuf)

        def body(j, carry):
            @pl.when(j >= 2)
            def _():
                wait(ssem)
            @pl.when(j + 4 < k)
            def _():
                gather(j + 4, lax.rem(j + 4, nbuf))
            wait(gsem)
            scatter(j, lax.rem(j, nbuf))
            return carry

        lax.fori_loop(0, k, body, 0)
        for _ in range(min(2, k)):
            wait(ssem)
        plsc.subcore_barrier()
        pltpu.sync_copy(acc.at[pl.ds(row0, rows_per_sub)],
                        out_hbm.at[c, pl.ds(row0, rows_per_sub)])

    return sc_kernel


def kernel(x_prev, x_same, x_next, edge_index, ln_gamma, ln_beta,
           W_root, W_neigh, b):
    n, d_prev = x_prev.shape
    d_out = W_root.shape[1]
    dh = d_out // 2
    e = edge_index.shape[1]

    k = -(-e // (_NS * _CH))
    e_pad = _NS * _CH * k
    ny = n + 16
    n_pad = -(-(n + 1) // (_NS * 8)) * (_NS * 8)

    bn = 2000
    grid = (n // bn,)
    root, y = pl.pallas_call(
        _dense_body,
        grid=grid,
        in_specs=[
            pl.BlockSpec((bn, d_prev), lambda i: (i, 0)),
            pl.BlockSpec((bn, d_prev), lambda i: (i, 0)),
            pl.BlockSpec((1, d_prev), lambda i: (0, 0)),
            pl.BlockSpec((1, d_prev), lambda i: (0, 0)),
            pl.BlockSpec(W_root.shape, lambda i: (0, 0)),
            pl.BlockSpec(W_neigh.shape, lambda i: (0, 0)),
            pl.BlockSpec((1, d_out), lambda i: (0, 0)),
        ],
        out_specs=[
            pl.BlockSpec((bn, d_out), lambda i: (i, 0)),
            pl.BlockSpec((2, bn, dh), lambda i: (0, i, 0)),
        ],
        out_shape=[
            jax.ShapeDtypeStruct((n, d_out), jnp.float32),
            jax.ShapeDtypeStruct((2, ny, dh), jnp.float32),
        ],
    )(x_prev, x_next, ln_gamma.reshape(1, -1), ln_beta.reshape(1, -1),
      W_root, W_neigh, b.reshape(1, -1))

    npad_e = e_pad - e
    pad_src = jnp.full((1, npad_e), n, jnp.int32)
    pad_dst = (n + jnp.arange(npad_e, dtype=jnp.int32) % (n_pad - n))[None]
    ei = jnp.concatenate([edge_index, jnp.concatenate([pad_src, pad_dst], 0)],
                         axis=1)
    ei = ei.reshape(2, _NS, k, _CH)
    zeros = jnp.zeros((n_pad, dh), jnp.float32)

    partials = _make_sc_kernel(n_pad, k, dh)(y, ei, zeros)

    p0 = partials[0, :n]
    p1 = partials[1, :n]
    out = pl.pallas_call(
        _combine_body,
        grid=grid,
        in_specs=[
            pl.BlockSpec((bn, d_out), lambda i: (i, 0)),
            pl.BlockSpec((bn, dh), lambda i: (i, 0)),
            pl.BlockSpec((bn, dh), lambda i: (i, 0)),
        ],
        out_specs=pl.BlockSpec((bn, d_out), lambda i: (i, 0)),
        out_shape=jax.ShapeDtypeStruct((n, d_out), jnp.float32),
    )(root, p0, p1)
    return out


# speedup vs baseline: 1.7875x; 1.0016x over previous
import functools

import jax
import jax.numpy as jnp
from jax import lax
from jax.experimental import pallas as pl
from jax.experimental.pallas import tpu as pltpu
from jax.experimental.pallas import tpu_sc as plsc

_LN_EPS = 1e-5
_CH = 128
_NC = 2
_NS = 16


def _dense_body(xp_ref, xn_ref, g_ref, bt_ref, wr_ref, wn_ref, b_ref,
                root_ref, y_ref):
    g = g_ref[...]
    bt = bt_ref[...]

    def ln(v):
        mu = jnp.mean(v, axis=-1, keepdims=True)
        var = jnp.mean((v - mu) * (v - mu), axis=-1, keepdims=True)
        return (v - mu) * lax.rsqrt(var + _LN_EPS) * g + bt

    x = jnp.concatenate([ln(xp_ref[...]), ln(xn_ref[...])], axis=1)
    root_ref[...] = (
        jnp.dot(x, wr_ref[...], preferred_element_type=jnp.float32) + b_ref[...]
    )
    y = jnp.dot(x, wn_ref[...], preferred_element_type=jnp.float32)
    d_half = y.shape[1] // 2
    y_ref[0] = y[:, :d_half]
    y_ref[1] = y[:, d_half:]


def _combine_body(root_ref, p0_ref, p1_ref, out_ref):
    agg = jnp.concatenate([p0_ref[...], p1_ref[...]], axis=1)
    out_ref[...] = root_ref[...] + agg


def _make_sc_kernel(n_pad, k, dh):
    rows_per_sub = n_pad // _NS
    mesh = plsc.VectorSubcoreMesh(core_axis_name="c", subcore_axis_name="s")
    nbuf = 6

    @functools.partial(
        pl.kernel,
        out_type=jax.ShapeDtypeStruct((_NC, n_pad, dh), jnp.float32),
        mesh=mesh,
        scratch_types=[
            pltpu.VMEM((k, _CH), jnp.int32),
            pltpu.VMEM((k, _CH), jnp.int32),
            pltpu.VMEM((nbuf, _CH, dh), jnp.float32),
            pltpu.VMEM_SHARED((n_pad, dh), jnp.float32),
            pltpu.SemaphoreType.DMA,
            pltpu.SemaphoreType.DMA,
        ],
        compiler_params=pltpu.CompilerParams(use_tc_tiling_on_sc=False),
    )
    def sc_kernel(y_hbm, ei_hbm, zeros_hbm, out_hbm,
                  src_v, dst_v, rows_v, acc, gsem, ssem):
        c = lax.axis_index("c")
        s = lax.axis_index("s")
        pltpu.sync_copy(ei_hbm.at[0, s], src_v)
        pltpu.sync_copy(ei_hbm.at[1, s], dst_v)
        row0 = s * rows_per_sub
        pltpu.sync_copy(zeros_hbm.at[pl.ds(row0, rows_per_sub)],
                        acc.at[pl.ds(row0, rows_per_sub)])
        plsc.subcore_barrier()

        def gather(j, slot):
            pltpu.async_copy(y_hbm.at[c].at[src_v.at[j]], rows_v.at[slot],
                             gsem)

        def scatter(j, slot):
            pltpu.async_copy(rows_v.at[slot], acc.at[dst_v.at[j]], ssem,
                             add=True)

        def wait(sem):
            pltpu.make_async_copy(zeros_hbm.at[pl.ds(0, _CH)],
                                  rows_v.at[0], sem).wait()

        for j in range(min(3, k)):
            gather(j, j % nbuf)

        def body(j, carry):
            @pl.when(j >= 3)
            def _():
                wait(ssem)
            @pl.when(j + 3 < k)
            def _():
                gather(j + 3, lax.rem(j + 3, nbuf))
            wait(gsem)
            scatter(j, lax.rem(j, nbuf))
            return carry

        lax.fori_loop(0, k, body, 0)
        for _ in range(min(3, k)):
            wait(ssem)
        plsc.subcore_barrier()
        pltpu.sync_copy(acc.at[pl.ds(row0, rows_per_sub)],
                        out_hbm.at[c, pl.ds(row0, rows_per_sub)])

    return sc_kernel


def kernel(x_prev, x_same, x_next, edge_index, ln_gamma, ln_beta,
           W_root, W_neigh, b):
    n, d_prev = x_prev.shape
    d_out = W_root.shape[1]
    dh = d_out // 2
    e = edge_index.shape[1]

    k = -(-e // (_NS * _CH))
    e_pad = _NS * _CH * k
    ny = n + 16
    n_pad = -(-(n + 1) // (_NS * 8)) * (_NS * 8)

    bn = 2000
    grid = (n // bn,)
    root, y = pl.pallas_call(
        _dense_body,
        grid=grid,
        in_specs=[
            pl.BlockSpec((bn, d_prev), lambda i: (i, 0)),
            pl.BlockSpec((bn, d_prev), lambda i: (i, 0)),
            pl.BlockSpec((1, d_prev), lambda i: (0, 0)),
            pl.BlockSpec((1, d_prev), lambda i: (0, 0)),
            pl.BlockSpec(W_root.shape, lambda i: (0, 0)),
            pl.BlockSpec(W_neigh.shape, lambda i: (0, 0)),
            pl.BlockSpec((1, d_out), lambda i: (0, 0)),
        ],
        out_specs=[
            pl.BlockSpec((bn, d_out), lambda i: (i, 0)),
            pl.BlockSpec((2, bn, dh), lambda i: (0, i, 0)),
        ],
        out_shape=[
            jax.ShapeDtypeStruct((n, d_out), jnp.float32),
            jax.ShapeDtypeStruct((2, ny, dh), jnp.float32),
        ],
    )(x_prev, x_next, ln_gamma.reshape(1, -1), ln_beta.reshape(1, -1),
      W_root, W_neigh, b.reshape(1, -1))

    npad_e = e_pad - e
    pad_src = jnp.full((1, npad_e), n, jnp.int32)
    pad_dst = (n + jnp.arange(npad_e, dtype=jnp.int32) % (n_pad - n))[None]
    ei = jnp.concatenate([edge_index, jnp.concatenate([pad_src, pad_dst], 0)],
                         axis=1)
    ei = ei.reshape(2, _NS, k, _CH)
    zeros = jnp.zeros((n_pad, dh), jnp.float32)

    partials = _make_sc_kernel(n_pad, k, dh)(y, ei, zeros)

    p0 = partials[0, :n]
    p1 = partials[1, :n]
    out = pl.pallas_call(
        _combine_body,
        grid=grid,
        in_specs=[
            pl.BlockSpec((bn, d_out), lambda i: (i, 0)),
            pl.BlockSpec((bn, dh), lambda i: (i, 0)),
            pl.BlockSpec((bn, dh), lambda i: (i, 0)),
        ],
        out_specs=pl.BlockSpec((bn, d_out), lambda i: (i, 0)),
        out_shape=jax.ShapeDtypeStruct((n, d_out), jnp.float32),
    )(root, p0, p1)
    return out
